# baseline (speedup 1.0000x reference)
import functools

import jax
import jax.numpy as jnp
from jax import lax
from jax.experimental import pallas as pl
from jax.experimental.pallas import tpu as pltpu
from jax.experimental.pallas import tpu_sc as plsc

BATCH = 16384
EMBED_D = 64
HIDDEN = 256

_NC = 2
_NS = 16
_NW = _NC * _NS
_BPW = BATCH // _NW
_CHUNK = 128
_NCHUNK = _BPW // _CHUNK


def _make_sc_gather():
    mesh = plsc.VectorSubcoreMesh(core_axis_name="c", subcore_axis_name="s")

    @functools.partial(
        pl.kernel,
        mesh=mesh,
        compiler_params=pltpu.CompilerParams(use_tc_tiling_on_sc=False),
        out_type=[
            jax.ShapeDtypeStruct((BATCH, EMBED_D), jnp.float32),
            jax.ShapeDtypeStruct((BATCH, EMBED_D), jnp.float32),
        ],
        scratch_types=[
            pltpu.VMEM((_BPW,), jnp.int32),
            pltpu.VMEM((_BPW, EMBED_D), jnp.float32),
            pltpu.VMEM((_BPW,), jnp.int32),
            pltpu.VMEM((_BPW, EMBED_D), jnp.float32),
            pltpu.SemaphoreType.DMA,
            pltpu.SemaphoreType.DMA,
        ],
    )
    def gather_kernel(users_hbm, movies_hbm, utab_hbm, mtab_hbm,
                      uout_hbm, mout_hbm,
                      uidx_v, urows_v, midx_v, mrows_v, sem_u, sem_m):
        wid = lax.axis_index("s") * _NC + lax.axis_index("c")
        base = wid * _BPW
        pltpu.sync_copy(users_hbm.at[pl.ds(base, _BPW)], uidx_v)
        pltpu.sync_copy(movies_hbm.at[pl.ds(base, _BPW)], midx_v)
        copies = []
        for j in range(_NCHUNK):
            sl = pl.ds(j * _CHUNK, _CHUNK)
            copies.append(pltpu.async_copy(
                utab_hbm.at[uidx_v.at[sl]], urows_v.at[sl], sem_u))
            copies.append(pltpu.async_copy(
                mtab_hbm.at[midx_v.at[sl]], mrows_v.at[sl], sem_m))
        for c in copies:
            c.wait()
        pltpu.sync_copy(urows_v, uout_hbm.at[pl.ds(base, _BPW)])
        pltpu.sync_copy(mrows_v, mout_hbm.at[pl.ds(base, _BPW)])

    return gather_kernel


_sc_gather = _make_sc_gather()

_ROWS_BLK = 1024


def _mlp_body(u_ref, m_ref, w1_ref, w2_ref, b_ref, o_ref):
    acc = jnp.dot(u_ref[...], w1_ref[...], preferred_element_type=jnp.float32)
    acc = acc + jnp.dot(m_ref[...], w2_ref[...],
                        preferred_element_type=jnp.float32)
    acc = acc + b_ref[...]
    o_ref[...] = jnp.maximum(acc, 0.0)


def _mlp(u_rows, m_rows, W, b):
    w1 = W[:EMBED_D]
    w2 = W[EMBED_D:]
    b2 = b.reshape(1, HIDDEN)
    grid = (BATCH // _ROWS_BLK,)
    return pl.pallas_call(
        _mlp_body,
        grid=grid,
        in_specs=[
            pl.BlockSpec((_ROWS_BLK, EMBED_D), lambda i: (i, 0)),
            pl.BlockSpec((_ROWS_BLK, EMBED_D), lambda i: (i, 0)),
            pl.BlockSpec((EMBED_D, HIDDEN), lambda i: (0, 0)),
            pl.BlockSpec((EMBED_D, HIDDEN), lambda i: (0, 0)),
            pl.BlockSpec((1, HIDDEN), lambda i: (0, 0)),
        ],
        out_specs=pl.BlockSpec((_ROWS_BLK, HIDDEN), lambda i: (i, 0)),
        out_shape=jax.ShapeDtypeStruct((BATCH, HIDDEN), jnp.float32),
    )(u_rows, m_rows, w1, w2, b2)


@jax.jit
def kernel(users, movies, user_table, movie_table, W, b):
    users = users.astype(jnp.int32)
    movies = movies.astype(jnp.int32)
    u_rows, m_rows = _sc_gather(users, movies, user_table, movie_table)
    return _mlp(u_rows, m_rows, W, b)


# speedup vs baseline: 1.5015x; 1.5015x over previous
import functools

import jax
import jax.numpy as jnp
from jax import lax
from jax.experimental import pallas as pl
from jax.experimental.pallas import tpu as pltpu
from jax.experimental.pallas import tpu_sc as plsc

BATCH = 16384
EMBED_D = 64
HIDDEN = 256

_NC = 2
_NS = 16
_NW = _NC * _NS
_BPW = BATCH // _NW
_FIRE = 16


def _make_sc_gather():
    mesh = plsc.VectorSubcoreMesh(core_axis_name="c", subcore_axis_name="s")

    @functools.partial(
        pl.kernel,
        mesh=mesh,
        out_type=[
            jax.ShapeDtypeStruct((BATCH, EMBED_D), jnp.float32),
            jax.ShapeDtypeStruct((BATCH, EMBED_D), jnp.float32),
        ],
        scratch_types=[
            pltpu.VMEM((_BPW,), jnp.int32),
            pltpu.VMEM((_BPW, EMBED_D), jnp.float32),
            pltpu.SemaphoreType.DMA,
        ],
    )
    def gather_kernel(users_hbm, movies_hbm, utab_hbm, mtab_hbm,
                      uout_hbm, mout_hbm,
                      idx_v, rows_v, sem):
        wid = lax.axis_index("s") * _NC + lax.axis_index("c")
        base = wid * _BPW

        def gather_one(src_idx_hbm, tab_hbm, out_hbm):
            pltpu.sync_copy(src_idx_hbm.at[pl.ds(base, _BPW)], idx_v)

            @pl.loop(0, _BPW, step=_FIRE)
            def _(i0):
                vec = idx_v[pl.ds(i0, _FIRE)]
                copies = []
                for j in range(_FIRE):
                    idx = vec[j]
                    copies.append(pltpu.async_copy(
                        tab_hbm.at[pl.ds(idx, 1), :],
                        rows_v.at[pl.ds(i0 + j, 1), :],
                        sem,
                    ))
                for c in copies:
                    c.wait()

            pltpu.sync_copy(rows_v, out_hbm.at[pl.ds(base, _BPW)])

        gather_one(users_hbm, utab_hbm, uout_hbm)
        gather_one(movies_hbm, mtab_hbm, mout_hbm)

    return gather_kernel


_sc_gather = _make_sc_gather()

_ROWS_BLK = 1024


def _mlp_body(u_ref, m_ref, w1_ref, w2_ref, b_ref, o_ref):
    acc = jnp.dot(u_ref[...], w1_ref[...], preferred_element_type=jnp.float32)
    acc = acc + jnp.dot(m_ref[...], w2_ref[...],
                        preferred_element_type=jnp.float32)
    acc = acc + b_ref[...]
    o_ref[...] = jnp.maximum(acc, 0.0)


def _mlp(u_rows, m_rows, W, b):
    w1 = W[:EMBED_D]
    w2 = W[EMBED_D:]
    b2 = b.reshape(1, HIDDEN)
    grid = (BATCH // _ROWS_BLK,)
    return pl.pallas_call(
        _mlp_body,
        grid=grid,
        in_specs=[
            pl.BlockSpec((_ROWS_BLK, EMBED_D), lambda i: (i, 0)),
            pl.BlockSpec((_ROWS_BLK, EMBED_D), lambda i: (i, 0)),
            pl.BlockSpec((EMBED_D, HIDDEN), lambda i: (0, 0)),
            pl.BlockSpec((EMBED_D, HIDDEN), lambda i: (0, 0)),
            pl.BlockSpec((1, HIDDEN), lambda i: (0, 0)),
        ],
        out_specs=pl.BlockSpec((_ROWS_BLK, HIDDEN), lambda i: (i, 0)),
        out_shape=jax.ShapeDtypeStruct((BATCH, HIDDEN), jnp.float32),
    )(u_rows, m_rows, w1, w2, b2)


@jax.jit
def kernel(users, movies, user_table, movie_table, W, b):
    users = users.astype(jnp.int32)
    movies = movies.astype(jnp.int32)
    u_rows, m_rows = _sc_gather(users, movies, user_table, movie_table)
    return _mlp(u_rows, m_rows, W, b)
